# baseline (speedup 1.0000x reference)
import functools

import jax
import jax.numpy as jnp
from jax import lax
from jax.experimental import pallas as pl
from jax.experimental.pallas import tpu as pltpu
from jax.experimental.pallas import tpu_sc as plsc

E = 64
TOP_K = 2
H = 1024
FF = 512
T = 2048
CAP = 128
PAD_ROW = E * CAP


def _build_dispatch(x, gw, gidx_s, ws_s, s0_ref, s1_ref):
    logits = lax.dot_general(x, gw, (((1,), (1,)), ((), ())),
                             preferred_element_type=jnp.float32)
    p = jax.nn.softmax(logits, axis=-1)

    eids = lax.broadcasted_iota(jnp.int32, (T, E), 1)
    m0 = jnp.max(p, axis=1, keepdims=True)
    idx0 = jnp.min(jnp.where(p >= m0, eids, E), axis=1, keepdims=True)
    oh0 = eids == idx0
    p1 = jnp.where(oh0, -jnp.inf, p)
    m1 = jnp.max(p1, axis=1, keepdims=True)
    idx1 = jnp.min(jnp.where(p1 >= m1, eids, E), axis=1, keepdims=True)
    oh1 = eids == idx1

    s = m0 + m1
    w0 = m0 / s
    w1 = m1 / s

    oh0f = oh0.astype(jnp.float32)
    oh1f = oh1.astype(jnp.float32)
    a = oh0f + oh1f

    chunk = 1024
    r = lax.broadcasted_iota(jnp.int32, (chunk, chunk), 0)
    c = lax.broadcasted_iota(jnp.int32, (chunk, chunk), 1)
    ltri = (r > c).astype(jnp.float32)
    carry = jnp.zeros((1, E), jnp.float32)
    parts = []
    for i in range(T // chunk):
        ai = lax.slice(a, (i * chunk, 0), ((i + 1) * chunk, E))
        ci = lax.dot_general(ltri, ai, (((1,), (0,)), ((), ())),
                             preferred_element_type=jnp.float32) + carry
        parts.append(ci)
        carry = carry + jnp.sum(ai, axis=0, keepdims=True)
    cum = jnp.concatenate(parts, axis=0)

    pos0 = jnp.sum(jnp.where(oh0, cum, 0.0), axis=1, keepdims=True)
    pos1 = jnp.sum(jnp.where(oh1, cum, 0.0), axis=1, keepdims=True)
    pos0i = pos0.astype(jnp.int32)
    pos1i = pos1.astype(jnp.int32)

    jidx = lax.broadcasted_iota(jnp.int32, (T, CAP), 1)
    ohj0 = (jidx == pos0i).astype(jnp.float32)
    ohj1 = (jidx == pos1i).astype(jnp.float32)

    ws_s[...] = (lax.dot_general(oh0f * w0, ohj0, (((0,), (0,)), ((), ())),
                                 preferred_element_type=jnp.float32) +
                 lax.dot_general(oh1f * w1, ohj1, (((0,), (0,)), ((), ())),
                                 preferred_element_type=jnp.float32))

    ti = lax.broadcasted_iota(jnp.int32, (T, 1), 0)
    thi = (ti // 16).astype(jnp.float32)
    tlo = (ti % 16).astype(jnp.float32)

    def _slotdot(v):
        return (lax.dot_general(oh0f * v, ohj0, (((0,), (0,)), ((), ())),
                                preferred_element_type=jnp.float32) +
                lax.dot_general(oh1f * v, ohj1, (((0,), (0,)), ((), ())),
                                preferred_element_type=jnp.float32))

    gidx = 16.0 * _slotdot(thi) + _slotdot(tlo)
    gidx_s[...] = jnp.round(gidx).astype(jnp.int32)

    s0_ref[...] = jnp.where(pos0i < CAP, idx0 * CAP + pos0i, PAD_ROW)
    s1_ref[...] = jnp.where(pos1i < CAP, idx1 * CAP + pos1i, PAD_ROW)


def _moe_body(x_ref, gw_ref, wg_ref, wu_ref, wd_ref,
              outd_ref, s0_ref, s1_ref, gidx_s, ws_s):
    e = pl.program_id(0)

    @pl.when(e == 0)
    def _():
        _build_dispatch(x_ref[...], gw_ref[...], gidx_s, ws_s, s0_ref, s1_ref)

    @pl.when(e < E)
    def _():
        gv = gidx_s[pl.ds(e, 1), :][0]
        wv = ws_s[pl.ds(e, 1), :][0]
        tids = lax.broadcasted_iota(jnp.int32, (CAP, T), 1)
        onehot = (tids == gv[:, None]).astype(jnp.float32)
        fast = lax.Precision.DEFAULT
        xb = lax.dot_general(onehot, x_ref[...], (((1,), (0,)), ((), ())),
                             precision=fast,
                             preferred_element_type=jnp.float32)
        g = lax.dot_general(xb, wg_ref[0], (((1,), (1,)), ((), ())),
                            precision=fast,
                            preferred_element_type=jnp.float32)
        u = lax.dot_general(xb, wu_ref[0], (((1,), (1,)), ((), ())),
                            precision=fast,
                            preferred_element_type=jnp.float32)
        sact = g * jax.nn.sigmoid(g) * u
        h = lax.dot_general(sact, wd_ref[0], (((1,), (1,)), ((), ())),
                            precision=fast,
                            preferred_element_type=jnp.float32)
        outd_ref[...] = h * wv[:, None]

    @pl.when(e >= E)
    def _():
        outd_ref[...] = jnp.zeros_like(outd_ref)


def _moe_tc(x, gate_w, w_gate, w_up, w_down):
    clamp = lambda e: jnp.minimum(e, E - 1)
    return pl.pallas_call(
        _moe_body,
        grid=(E + 1,),
        in_specs=[
            pl.BlockSpec((T, H), lambda e: (0, 0)),
            pl.BlockSpec((E, H), lambda e: (0, 0)),
            pl.BlockSpec((1, FF, H), lambda e: (clamp(e), 0, 0)),
            pl.BlockSpec((1, FF, H), lambda e: (clamp(e), 0, 0)),
            pl.BlockSpec((1, H, FF), lambda e: (clamp(e), 0, 0)),
        ],
        out_specs=(
            pl.BlockSpec((CAP, H), lambda e: (e, 0)),
            pl.BlockSpec((T, 1), lambda e: (0, 0)),
            pl.BlockSpec((T, 1), lambda e: (0, 0)),
        ),
        out_shape=(
            jax.ShapeDtypeStruct(((E + 1) * CAP, H), jnp.float32),
            jax.ShapeDtypeStruct((T, 1), jnp.int32),
            jax.ShapeDtypeStruct((T, 1), jnp.int32),
        ),
        scratch_shapes=[
            pltpu.VMEM((E, CAP), jnp.int32),
            pltpu.VMEM((E, CAP), jnp.float32),
        ],
        compiler_params=pltpu.CompilerParams(
            dimension_semantics=("arbitrary",)),
    )(x, gate_w, w_gate, w_up, w_down)


def _sc_combine(outd, slot0, slot1):
    info = plsc.get_sparse_core_info()
    nc, ns = info.num_cores, info.num_subcores
    nw = nc * ns
    tok_per_w = T // nw
    chunk = 32

    mesh = plsc.VectorSubcoreMesh(core_axis_name="c", subcore_axis_name="s")

    @functools.partial(
        pl.kernel, mesh=mesh,
        out_type=jax.ShapeDtypeStruct((T, H), jnp.float32),
        scratch_types=[
            pltpu.VMEM((tok_per_w,), jnp.int32),
            pltpu.VMEM((tok_per_w,), jnp.int32),
            pltpu.VMEM((chunk, H), jnp.float32),
            pltpu.VMEM((chunk, H), jnp.float32),
            pltpu.VMEM((chunk, H), jnp.float32),
            pltpu.SemaphoreType.DMA,
        ],
    )
    def k(outd_hbm, s0_hbm, s1_hbm, out_hbm, i0_v, i1_v, b0, b1, ob, sem):
        wid = lax.axis_index("s") * nc + lax.axis_index("c")
        base = wid * tok_per_w
        pltpu.sync_copy(s0_hbm.at[pl.ds(base, tok_per_w)], i0_v)
        pltpu.sync_copy(s1_hbm.at[pl.ds(base, tok_per_w)], i1_v)
        for ci in range(tok_per_w // chunk):
            pltpu.async_copy(outd_hbm.at[i0_v.at[pl.ds(ci * chunk, chunk)]],
                             b0, sem).wait()
            pltpu.async_copy(outd_hbm.at[i1_v.at[pl.ds(ci * chunk, chunk)]],
                             b1, sem).wait()

            def row(rr, _):
                for v in range(H // 16):
                    sl = pl.ds(v * 16, 16)
                    ob[rr, sl] = b0[rr, sl] + b1[rr, sl]
                return 0

            lax.fori_loop(0, chunk, row, 0)
            pltpu.sync_copy(ob, out_hbm.at[pl.ds(base + ci * chunk, chunk)])

    return k(outd, slot0, slot1)


def kernel(hidden_states, gate_w, w_gate, w_up, w_down):
    outd, slot0, slot1 = _moe_tc(hidden_states, gate_w, w_gate, w_up, w_down)
    return _sc_combine(outd, slot0.reshape(T), slot1.reshape(T))


# speedup vs baseline: 1.0029x; 1.0029x over previous
import functools

import jax
import jax.numpy as jnp
from jax import lax
from jax.experimental import pallas as pl
from jax.experimental.pallas import tpu as pltpu
from jax.experimental.pallas import tpu_sc as plsc

E = 64
TOP_K = 2
H = 1024
FF = 512
T = 2048
CAP = 128
PAD_ROW = E * CAP


def _build_dispatch(x, gw, gidx_s, ws_s, s0_ref, s1_ref):
    logits = lax.dot_general(x, gw, (((1,), (1,)), ((), ())),
                             preferred_element_type=jnp.float32)
    p = jax.nn.softmax(logits, axis=-1)

    eids = lax.broadcasted_iota(jnp.int32, (T, E), 1)
    m0 = jnp.max(p, axis=1, keepdims=True)
    idx0 = jnp.min(jnp.where(p >= m0, eids, E), axis=1, keepdims=True)
    oh0 = eids == idx0
    p1 = jnp.where(oh0, -jnp.inf, p)
    m1 = jnp.max(p1, axis=1, keepdims=True)
    idx1 = jnp.min(jnp.where(p1 >= m1, eids, E), axis=1, keepdims=True)
    oh1 = eids == idx1

    s = m0 + m1
    w0 = m0 / s
    w1 = m1 / s

    oh0f = oh0.astype(jnp.float32)
    oh1f = oh1.astype(jnp.float32)
    a = oh0f + oh1f

    chunk = 1024
    r = lax.broadcasted_iota(jnp.int32, (chunk, chunk), 0)
    c = lax.broadcasted_iota(jnp.int32, (chunk, chunk), 1)
    ltri = (r > c).astype(jnp.float32)
    carry = jnp.zeros((1, E), jnp.float32)
    parts = []
    for i in range(T // chunk):
        ai = lax.slice(a, (i * chunk, 0), ((i + 1) * chunk, E))
        ci = lax.dot_general(ltri, ai, (((1,), (0,)), ((), ())),
                             preferred_element_type=jnp.float32) + carry
        parts.append(ci)
        carry = carry + jnp.sum(ai, axis=0, keepdims=True)
    cum = jnp.concatenate(parts, axis=0)

    pos0 = jnp.sum(jnp.where(oh0, cum, 0.0), axis=1, keepdims=True)
    pos1 = jnp.sum(jnp.where(oh1, cum, 0.0), axis=1, keepdims=True)
    pos0i = pos0.astype(jnp.int32)
    pos1i = pos1.astype(jnp.int32)

    jidx = lax.broadcasted_iota(jnp.int32, (T, CAP), 1)
    ohj0 = (jidx == pos0i).astype(jnp.float32)
    ohj1 = (jidx == pos1i).astype(jnp.float32)

    ws_s[...] = (lax.dot_general(oh0f * w0, ohj0, (((0,), (0,)), ((), ())),
                                 preferred_element_type=jnp.float32) +
                 lax.dot_general(oh1f * w1, ohj1, (((0,), (0,)), ((), ())),
                                 preferred_element_type=jnp.float32))

    ti = lax.broadcasted_iota(jnp.int32, (T, 1), 0)
    thi = (ti // 16).astype(jnp.float32)
    tlo = (ti % 16).astype(jnp.float32)

    def _slotdot(v):
        return (lax.dot_general(oh0f * v, ohj0, (((0,), (0,)), ((), ())),
                                preferred_element_type=jnp.float32) +
                lax.dot_general(oh1f * v, ohj1, (((0,), (0,)), ((), ())),
                                preferred_element_type=jnp.float32))

    gidx = 16.0 * _slotdot(thi) + _slotdot(tlo)
    gidx_s[...] = jnp.round(gidx).astype(jnp.int32)

    s0_ref[...] = jnp.where(pos0i < CAP, idx0 * CAP + pos0i, PAD_ROW)
    s1_ref[...] = jnp.where(pos1i < CAP, idx1 * CAP + pos1i, PAD_ROW)


def _moe_body(x_ref, gw_ref, wg_ref, wu_ref, wd_ref,
              outd_ref, s0_ref, s1_ref, gidx_s, ws_s):
    e = pl.program_id(0)

    @pl.when(e == 0)
    def _():
        _build_dispatch(x_ref[...], gw_ref[...], gidx_s, ws_s, s0_ref, s1_ref)

    @pl.when(e < E)
    def _():
        gv = gidx_s[pl.ds(e, 1), :][0]
        wv = ws_s[pl.ds(e, 1), :][0]
        tids = lax.broadcasted_iota(jnp.int32, (CAP, T), 1)
        onehot = (tids == gv[:, None]).astype(jnp.float32)
        xb = lax.dot_general(onehot, x_ref[...], (((1,), (0,)), ((), ())),
                             preferred_element_type=jnp.float32)
        g = lax.dot_general(xb, wg_ref[0], (((1,), (1,)), ((), ())),
                            preferred_element_type=jnp.float32)
        u = lax.dot_general(xb, wu_ref[0], (((1,), (1,)), ((), ())),
                            preferred_element_type=jnp.float32)
        sact = g * jax.nn.sigmoid(g) * u
        h = lax.dot_general(sact, wd_ref[0], (((1,), (1,)), ((), ())),
                            preferred_element_type=jnp.float32)
        outd_ref[...] = h * wv[:, None]

    @pl.when(e >= E)
    def _():
        outd_ref[...] = jnp.zeros_like(outd_ref)


def _moe_tc(x, gate_w, w_gate, w_up, w_down):
    clamp = lambda e: jnp.minimum(e, E - 1)
    return pl.pallas_call(
        _moe_body,
        grid=(E + 1,),
        in_specs=[
            pl.BlockSpec((T, H), lambda e: (0, 0)),
            pl.BlockSpec((E, H), lambda e: (0, 0)),
            pl.BlockSpec((1, FF, H), lambda e: (clamp(e), 0, 0)),
            pl.BlockSpec((1, FF, H), lambda e: (clamp(e), 0, 0)),
            pl.BlockSpec((1, H, FF), lambda e: (clamp(e), 0, 0)),
        ],
        out_specs=(
            pl.BlockSpec((CAP, H), lambda e: (e, 0)),
            pl.BlockSpec((T, 1), lambda e: (0, 0)),
            pl.BlockSpec((T, 1), lambda e: (0, 0)),
        ),
        out_shape=(
            jax.ShapeDtypeStruct(((E + 1) * CAP, H), jnp.float32),
            jax.ShapeDtypeStruct((T, 1), jnp.int32),
            jax.ShapeDtypeStruct((T, 1), jnp.int32),
        ),
        scratch_shapes=[
            pltpu.VMEM((E, CAP), jnp.int32),
            pltpu.VMEM((E, CAP), jnp.float32),
        ],
        compiler_params=pltpu.CompilerParams(
            dimension_semantics=("arbitrary",)),
    )(x, gate_w, w_gate, w_up, w_down)


def _sc_combine(outd, slot0, slot1):
    info = plsc.get_sparse_core_info()
    nc, ns = info.num_cores, info.num_subcores
    nw = nc * ns
    tok_per_w = T // nw
    chunk = 32

    mesh = plsc.VectorSubcoreMesh(core_axis_name="c", subcore_axis_name="s")

    @functools.partial(
        pl.kernel, mesh=mesh,
        out_type=jax.ShapeDtypeStruct((T, H), jnp.float32),
        scratch_types=[
            pltpu.VMEM((tok_per_w,), jnp.int32),
            pltpu.VMEM((tok_per_w,), jnp.int32),
            pltpu.VMEM((chunk, H), jnp.float32),
            pltpu.VMEM((chunk, H), jnp.float32),
            pltpu.VMEM((chunk, H), jnp.float32),
            pltpu.SemaphoreType.DMA,
        ],
    )
    def k(outd_hbm, s0_hbm, s1_hbm, out_hbm, i0_v, i1_v, b0, b1, ob, sem):
        wid = lax.axis_index("s") * nc + lax.axis_index("c")
        base = wid * tok_per_w
        pltpu.sync_copy(s0_hbm.at[pl.ds(base, tok_per_w)], i0_v)
        pltpu.sync_copy(s1_hbm.at[pl.ds(base, tok_per_w)], i1_v)
        for ci in range(tok_per_w // chunk):
            pltpu.async_copy(outd_hbm.at[i0_v.at[pl.ds(ci * chunk, chunk)]],
                             b0, sem).wait()
            pltpu.async_copy(outd_hbm.at[i1_v.at[pl.ds(ci * chunk, chunk)]],
                             b1, sem).wait()

            def row(rr, _):
                for v in range(H // 16):
                    sl = pl.ds(v * 16, 16)
                    ob[rr, sl] = b0[rr, sl] + b1[rr, sl]
                return 0

            lax.fori_loop(0, chunk, row, 0)
            pltpu.sync_copy(ob, out_hbm.at[pl.ds(base + ci * chunk, chunk)])

    return k(outd, slot0, slot1)


def kernel(hidden_states, gate_w, w_gate, w_up, w_down):
    outd, slot0, slot1 = _moe_tc(hidden_states, gate_w, w_gate, w_up, w_down)
    return _sc_combine(outd, slot0.reshape(T), slot1.reshape(T))
